# initial kernel scaffold (unmeasured)
import jax
import jax.numpy as jnp
from jax import lax
from jax.experimental import pallas as pl
from jax.experimental.pallas import tpu as pltpu

N_DEV = 32


def kernel(x, w_mat):
    m_total, k_blk = x.shape
    k_total, n_dim = w_mat.shape
    m_blk = m_total // N_DEV

    def body(x_ref, w_ref, out_ref, xg_ref, send_sems, recv_sems):
        me = lax.axis_index("i")

        barrier_sem = pltpu.get_barrier_semaphore()
        for d in range(1, N_DEV):
            peer = lax.rem(me + d, N_DEV)
            pl.semaphore_signal(
                barrier_sem, inc=1,
                device_id=(peer,), device_id_type=pl.DeviceIdType.MESH,
            )
        pl.semaphore_wait(barrier_sem, N_DEV - 1)

        xg_ref[:, pl.ds(me * k_blk, k_blk)] = x_ref[pl.ds(me * m_blk, m_blk), :]

        rdmas = []
        for d in range(1, N_DEV):
            tgt = lax.rem(me + d, N_DEV)
            rdma = pltpu.make_async_remote_copy(
                src_ref=x_ref.at[pl.ds(tgt * m_blk, m_blk), :],
                dst_ref=xg_ref.at[:, pl.ds(me * k_blk, k_blk)],
                send_sem=send_sems.at[d - 1],
                recv_sem=recv_sems.at[d - 1],
                device_id=(tgt,),
                device_id_type=pl.DeviceIdType.MESH,
            )
            rdma.start()
            rdmas.append(rdma)

        for r in rdmas:
            r.wait_recv()
        for r in rdmas:
            r.wait_send()

        acc = jnp.dot(xg_ref[:, :], w_ref[:, :],
                      preferred_element_type=jnp.float32)
        out_ref[:, :] = jnp.maximum(acc, 0.0)

    return pl.pallas_call(
        body,
        out_shape=jax.ShapeDtypeStruct((m_blk, n_dim), jnp.float32),
        in_specs=[
            pl.BlockSpec(memory_space=pltpu.VMEM),
            pl.BlockSpec(memory_space=pltpu.VMEM),
        ],
        out_specs=pl.BlockSpec(memory_space=pltpu.VMEM),
        scratch_shapes=[
            pltpu.VMEM((m_blk, k_total), jnp.float32),
            pltpu.SemaphoreType.DMA((N_DEV - 1,)),
            pltpu.SemaphoreType.DMA((N_DEV - 1,)),
        ],
        compiler_params=pltpu.CompilerParams(collective_id=0),
    )(x, w_mat)


# baseline (device time: 49549 ns/iter reference)
import jax
import jax.numpy as jnp
from jax import lax
from jax.experimental import pallas as pl
from jax.experimental.pallas import tpu as pltpu

N_DEV = 32


def kernel(x, w_mat):
    m_total, k_blk = x.shape
    k_total, n_dim = w_mat.shape
    m_blk = m_total // N_DEV

    def body(x_ref, w_ref, out_ref, xg_ref, send_sems, recv_sems):
        me = lax.axis_index("i")

        barrier_sem = pltpu.get_barrier_semaphore()
        for d in range(1, N_DEV):
            peer = lax.rem(me + d, N_DEV)
            pl.semaphore_signal(
                barrier_sem, inc=1,
                device_id=(peer,), device_id_type=pl.DeviceIdType.MESH,
            )
        pl.semaphore_wait(barrier_sem, N_DEV - 1)

        xg_ref[:, pl.ds(me * k_blk, k_blk)] = x_ref[pl.ds(me * m_blk, m_blk), :]

        rdmas = []
        for d in range(1, N_DEV):
            tgt = lax.rem(me + d, N_DEV)
            rdma = pltpu.make_async_remote_copy(
                src_ref=x_ref.at[pl.ds(tgt * m_blk, m_blk), :],
                dst_ref=xg_ref.at[:, pl.ds(me * k_blk, k_blk)],
                send_sem=send_sems.at[d - 1],
                recv_sem=recv_sems.at[d - 1],
                device_id=(tgt,),
                device_id_type=pl.DeviceIdType.MESH,
            )
            rdma.start()
            rdmas.append(rdma)

        for r in rdmas:
            r.wait_recv()
        for r in rdmas:
            r.wait_send()

        acc = jnp.dot(xg_ref[:, :], w_ref[:, :],
                      preferred_element_type=jnp.float32)
        out_ref[:, :] = jnp.maximum(acc, 0.0)

    return pl.pallas_call(
        body,
        out_shape=jax.ShapeDtypeStruct((m_blk, n_dim), jnp.float32),
        in_specs=[
            pl.BlockSpec(memory_space=pltpu.VMEM),
            pl.BlockSpec(memory_space=pltpu.VMEM),
        ],
        out_specs=pl.BlockSpec(memory_space=pltpu.VMEM),
        scratch_shapes=[
            pltpu.VMEM((m_blk, k_total), jnp.float32),
            pltpu.SemaphoreType.DMA((N_DEV - 1,)),
            pltpu.SemaphoreType.DMA((N_DEV - 1,)),
        ],
        compiler_params=pltpu.CompilerParams(
            collective_id=0,
            vmem_limit_bytes=100 * 1024 * 1024,
        ),
    )(x, w_mat)


# device time: 48576 ns/iter; 1.0200x vs baseline; 1.0200x over previous
import jax
import jax.numpy as jnp
from jax import lax
from jax.experimental import pallas as pl
from jax.experimental.pallas import tpu as pltpu

N_DEV = 32


def kernel(x, w_mat):
    m_total, k_blk = x.shape
    k_total, n_dim = w_mat.shape
    m_blk = m_total // N_DEV

    def body(x_ref, w_ref, out_ref, xg_ref, send_sems, recv_sems):
        me = lax.axis_index("i")

        barrier_sem = pltpu.get_barrier_semaphore()
        for d in range(1, N_DEV):
            peer = lax.rem(me + d, N_DEV)
            pl.semaphore_signal(
                barrier_sem, inc=1,
                device_id=(peer,), device_id_type=pl.DeviceIdType.MESH,
            )
        pl.semaphore_wait(barrier_sem, N_DEV - 1)

        rdmas = []
        for d in range(1, N_DEV):
            tgt = lax.rem(me + d, N_DEV)
            rdma = pltpu.make_async_remote_copy(
                src_ref=x_ref.at[pl.ds(tgt * m_blk, m_blk), :],
                dst_ref=xg_ref.at[me],
                send_sem=send_sems.at[d - 1],
                recv_sem=recv_sems.at[d - 1],
                device_id=(tgt,),
                device_id_type=pl.DeviceIdType.MESH,
            )
            rdma.start()
            rdmas.append(rdma)

        acc = jnp.dot(
            x_ref[pl.ds(me * m_blk, m_blk), :],
            w_ref[pl.ds(me * k_blk, k_blk), :],
            preferred_element_type=jnp.float32,
        )

        for d in range(1, N_DEV):
            rdmas[d - 1].wait_recv()
            src_dev = lax.rem(me + (N_DEV - d), N_DEV)
            acc = acc + jnp.dot(
                xg_ref[src_dev],
                w_ref[pl.ds(src_dev * k_blk, k_blk), :],
                preferred_element_type=jnp.float32,
            )

        out_ref[:, :] = jnp.maximum(acc, 0.0)

        for r in rdmas:
            r.wait_send()

    return pl.pallas_call(
        body,
        out_shape=jax.ShapeDtypeStruct((m_blk, n_dim), jnp.float32),
        in_specs=[
            pl.BlockSpec(memory_space=pltpu.VMEM),
            pl.BlockSpec(memory_space=pltpu.VMEM),
        ],
        out_specs=pl.BlockSpec(memory_space=pltpu.VMEM),
        scratch_shapes=[
            pltpu.VMEM((N_DEV, m_blk, k_blk), jnp.float32),
            pltpu.SemaphoreType.DMA((N_DEV - 1,)),
            pltpu.SemaphoreType.DMA((N_DEV - 1,)),
        ],
        compiler_params=pltpu.CompilerParams(
            collective_id=0,
            vmem_limit_bytes=100 * 1024 * 1024,
        ),
    )(x, w_mat)


# device time: 41466 ns/iter; 1.1949x vs baseline; 1.1715x over previous
import jax
import jax.numpy as jnp
from jax import lax
from jax.experimental import pallas as pl
from jax.experimental.pallas import tpu as pltpu

N_DEV = 32


def kernel(x, w_mat):
    m_total, k_blk = x.shape
    k_total, n_dim = w_mat.shape
    m_blk = m_total // N_DEV

    def body(x_ref, w_ref, out_ref, xs_ref, wb_ref, xg_ref,
             send_sems, recv_sems, round_sems):
        me = lax.axis_index("i")

        xs_ref[:, :] = x_ref[:, :].astype(jnp.bfloat16)

        barrier_sem = pltpu.get_barrier_semaphore()
        for k in range(5):
            peer = lax.rem(me + 2**k, N_DEV)
            sem = barrier_sem if k == 0 else round_sems.at[k - 1]
            pl.semaphore_signal(
                sem, inc=1,
                device_id=(peer,), device_id_type=pl.DeviceIdType.MESH,
            )
            pl.semaphore_wait(sem, 1)

        rdmas = []
        for d in range(1, N_DEV):
            tgt = lax.rem(me + d, N_DEV)
            rdma = pltpu.make_async_remote_copy(
                src_ref=xs_ref.at[pl.ds(tgt * m_blk, m_blk), :],
                dst_ref=xg_ref.at[me],
                send_sem=send_sems.at[d - 1],
                recv_sem=recv_sems.at[d - 1],
                device_id=(tgt,),
                device_id_type=pl.DeviceIdType.MESH,
            )
            rdma.start()
            rdmas.append(rdma)

        wb_ref[:, :] = w_ref[:, :].astype(jnp.bfloat16)
        acc = jnp.dot(
            xs_ref[pl.ds(me * m_blk, m_blk), :],
            wb_ref[pl.ds(me * k_blk, k_blk), :],
            preferred_element_type=jnp.float32,
        )

        for d in range(1, N_DEV):
            rdmas[d - 1].wait_recv()
            src_dev = lax.rem(me + (N_DEV - d), N_DEV)
            acc = acc + jnp.dot(
                xg_ref[src_dev],
                wb_ref[pl.ds(src_dev * k_blk, k_blk), :],
                preferred_element_type=jnp.float32,
            )

        out_ref[:, :] = jnp.maximum(acc, 0.0)

        for r in rdmas:
            r.wait_send()

    return pl.pallas_call(
        body,
        out_shape=jax.ShapeDtypeStruct((m_blk, n_dim), jnp.float32),
        in_specs=[
            pl.BlockSpec(memory_space=pltpu.VMEM),
            pl.BlockSpec(memory_space=pltpu.VMEM),
        ],
        out_specs=pl.BlockSpec(memory_space=pltpu.VMEM),
        scratch_shapes=[
            pltpu.VMEM((m_total, k_blk), jnp.bfloat16),
            pltpu.VMEM((k_total, n_dim), jnp.bfloat16),
            pltpu.VMEM((N_DEV, m_blk, k_blk), jnp.bfloat16),
            pltpu.SemaphoreType.DMA((N_DEV - 1,)),
            pltpu.SemaphoreType.DMA((N_DEV - 1,)),
            pltpu.SemaphoreType.REGULAR((4,)),
        ],
        compiler_params=pltpu.CompilerParams(
            collective_id=0,
            vmem_limit_bytes=100 * 1024 * 1024,
        ),
    )(x, w_mat)


# device time: 38380 ns/iter; 1.2910x vs baseline; 1.0804x over previous
import jax
import jax.numpy as jnp
from jax import lax
from jax.experimental import pallas as pl
from jax.experimental.pallas import tpu as pltpu

N_DEV = 32


def kernel(x, w_mat):
    m_total, k_blk = x.shape
    k_total, n_dim = w_mat.shape
    m_blk = m_total // N_DEV

    def body(x_ref, w_ref, out_ref, xs_ref, xg_ref,
             send_sems, recv_sems, round_sems):
        me = lax.axis_index("i")

        xs_ref[:, :] = x_ref[:, :].astype(jnp.bfloat16)

        def send_to(d):
            tgt = lax.rem(me + d, N_DEV)
            rdma = pltpu.make_async_remote_copy(
                src_ref=xs_ref.at[pl.ds(tgt * m_blk, m_blk), :],
                dst_ref=xg_ref.at[me],
                send_sem=send_sems.at[d - 1],
                recv_sem=recv_sems.at[d - 1],
                device_id=(tgt,),
                device_id_type=pl.DeviceIdType.MESH,
            )
            rdma.start()
            return rdma

        barrier_sem = pltpu.get_barrier_semaphore()

        def round_sig(k):
            peer = lax.rem(me + (N_DEV - 2**k), N_DEV)
            sem = barrier_sem if k == 0 else round_sems.at[k - 1]
            pl.semaphore_signal(
                sem, inc=1,
                device_id=(peer,), device_id_type=pl.DeviceIdType.MESH,
            )

        rdmas = []
        round_sig(0)
        for k in range(5):
            pl.semaphore_wait(
                barrier_sem if k == 0 else round_sems.at[k - 1], 1
            )
            if k < 4:
                round_sig(k + 1)
            for d in range(2**k, min(2**(k + 1), N_DEV)):
                rdmas.append(send_to(d))

        accs = [None, None, None, None]
        accs[0] = jnp.dot(
            xs_ref[pl.ds(me * m_blk, m_blk), :].astype(jnp.float32),
            w_ref[pl.ds(me * k_blk, k_blk), :],
            preferred_element_type=jnp.float32,
        )

        for d in range(1, N_DEV):
            rdmas[d - 1].wait_recv()
            src_dev = lax.rem(me + (N_DEV - d), N_DEV)
            p = jnp.dot(
                xg_ref[src_dev].astype(jnp.float32),
                w_ref[pl.ds(src_dev * k_blk, k_blk), :],
                preferred_element_type=jnp.float32,
            )
            a = d % 4
            accs[a] = p if accs[a] is None else accs[a] + p

        acc = (accs[0] + accs[1]) + (accs[2] + accs[3])
        out_ref[:, :] = jnp.maximum(acc, 0.0)

        for r in rdmas:
            r.wait_send()

    return pl.pallas_call(
        body,
        out_shape=jax.ShapeDtypeStruct((m_blk, n_dim), jnp.float32),
        in_specs=[
            pl.BlockSpec(memory_space=pltpu.VMEM),
            pl.BlockSpec(memory_space=pltpu.VMEM),
        ],
        out_specs=pl.BlockSpec(memory_space=pltpu.VMEM),
        scratch_shapes=[
            pltpu.VMEM((m_total, k_blk), jnp.bfloat16),
            pltpu.VMEM((N_DEV, m_blk, k_blk), jnp.bfloat16),
            pltpu.SemaphoreType.DMA((N_DEV - 1,)),
            pltpu.SemaphoreType.DMA((N_DEV - 1,)),
            pltpu.SemaphoreType.REGULAR((4,)),
        ],
        compiler_params=pltpu.CompilerParams(
            collective_id=0,
            vmem_limit_bytes=100 * 1024 * 1024,
        ),
    )(x, w_mat)
